# Initial kernel scaffold; baseline (speedup 1.0000x reference)
import jax
import jax.numpy as jnp
from jax.experimental import pallas as pl


def kernel(F_all, edge_und, edge_dir, Wp1, bp1, Wp2, bp2, Win, b_in, Ws0, bs0, Wn0, bn0, Ws1, bs1, Wn1, bn1, Ws2, bs2, Wn2, bn2, We1, be1, We2, be2, We3, be3):
    raise NotImplementedError("write your pallas kernel here")


# speedup vs baseline: 1.0831x; 1.0831x over previous
import functools

import jax
import jax.numpy as jnp
from jax import lax
from jax.experimental import pallas as pl

N = 10000
E = 320000
D = 128
H = 128

_INTERPRET = False


def _dot(a, b):
    return jnp.dot(a, b, preferred_element_type=jnp.float32)


def _node_body(f_ref, wp1_ref, bp1_ref, wp2_ref, bp2_ref, winc_ref, winp_ref,
               bin_ref, ws_ref, bsn_ref, wn_ref,
               czp_ref, hs_ref, hw_ref):
    x = f_ref[...]
    n = x.shape[0]
    col = lax.broadcasted_iota(jnp.int32, (1, D), 1)
    mask = col < 3
    s1 = jnp.sum(x, axis=0, keepdims=True)
    s2 = jnp.sum(x * x, axis=0, keepdims=True)
    m = s1 / n
    var = (s2 - n * m * m) / (n - 1)
    std = jnp.sqrt(jnp.maximum(var, 0.0))
    rs = jnp.where(mask, 1.0 / (std + 1e-6), 0.0)
    mm = jnp.where(mask, m, 0.0)
    czf = (x - mm) * rs
    p1 = jax.nn.relu(_dot(x, wp1_ref[...]) + bp1_ref[...])
    p2 = jax.nn.relu(_dot(p1, wp2_ref[...]) + bp2_ref[...])
    h0 = jax.nn.relu(_dot(czf, winc_ref[...]) + _dot(p2, winp_ref[...])
                     + bin_ref[...])
    czp_ref[...] = czf[:, :16]
    hs_ref[...] = _dot(h0, ws_ref[...]) + bsn_ref[...]
    hw_ref[...] = _dot(h0, wn_ref[...])


def _node_stage(F_all, Wp1e, bp1, Wp2, bp2, Wince, Winp, b_in, Ws, bsn, Wn):
    return pl.pallas_call(
        _node_body,
        out_shape=(
            jax.ShapeDtypeStruct((N, 16), jnp.float32),
            jax.ShapeDtypeStruct((N, H), jnp.float32),
            jax.ShapeDtypeStruct((N, H), jnp.float32),
        ),
        interpret=_INTERPRET,
    )(F_all, Wp1e, bp1[None, :], Wp2, bp2[None, :], Wince, Winp,
      b_in[None, :], Ws, bsn[None, :], Wn)


def _comb_body(hs_ref, seg_ref, deg_ref, ws_ref, bsn_ref, wn_ref,
               hs2_ref, hw2_ref):
    d = deg_ref[...]
    inv = 1.0 / jnp.maximum(d, 1.0)
    h = jax.nn.relu(hs_ref[...] + seg_ref[...] * inv[:, :1])
    hs2_ref[...] = _dot(h, ws_ref[...]) + bsn_ref[...]
    hw2_ref[...] = _dot(h, wn_ref[...])


def _comb_stage(hs, seg, deg, Ws, bsn, Wn):
    return pl.pallas_call(
        _comb_body,
        out_shape=(
            jax.ShapeDtypeStruct((N, H), jnp.float32),
            jax.ShapeDtypeStruct((N, H), jnp.float32),
        ),
        interpret=_INTERPRET,
    )(hs, seg, deg, Ws, bsn, Wn)


def _comb_final_body(hs_ref, seg_ref, deg_ref, h_ref):
    d = deg_ref[...]
    inv = 1.0 / jnp.maximum(d, 1.0)
    h_ref[...] = jax.nn.relu(hs_ref[...] + seg_ref[...] * inv[:, :1])


def _comb_final(hs, seg, deg):
    return pl.pallas_call(
        _comb_final_body,
        out_shape=jax.ShapeDtypeStruct((N, H), jnp.float32),
        interpret=_INTERPRET,
    )(hs, seg, deg)


_EB = 2000


def _edge_body(hi_ref, hj_ref, czi_ref, czj_ref, w1a_ref, w1b_ref, w1c_ref,
               w1d_ref, be1_ref, w2_ref, be2_ref, w3_ref, be3_ref, out_ref):
    hi = hi_ref[...]
    hj = hj_ref[...]
    dz = czi_ref[...] - czj_ref[...]
    x = (_dot(hi, w1a_ref[...]) + _dot(hj, w1b_ref[...])
         + _dot(jnp.abs(hi - hj), w1c_ref[...]) + _dot(dz, w1d_ref[...]))
    x = jax.nn.relu(x + be1_ref[...])
    x = jax.nn.relu(_dot(x, w2_ref[...]) + be2_ref[...])
    out_ref[...] = jnp.sum(x * w3_ref[...], axis=1, keepdims=True) + be3_ref[...]


def _edge_stage(HI, HJ, CZI, CZJ, W1a, W1b, W1c, W1dp, be1, We2, be2, w3r, be3):
    grid = (E // _EB,)
    full = lambda shape: pl.BlockSpec(shape, lambda i: (0, 0))
    out = pl.pallas_call(
        _edge_body,
        grid=grid,
        in_specs=[
            pl.BlockSpec((_EB, H), lambda i: (i, 0)),
            pl.BlockSpec((_EB, H), lambda i: (i, 0)),
            pl.BlockSpec((_EB, 16), lambda i: (i, 0)),
            pl.BlockSpec((_EB, 16), lambda i: (i, 0)),
            full((H, 256)), full((H, 256)), full((H, 256)), full((16, 256)),
            full((1, 256)), full((256, 256)), full((1, 256)), full((1, 256)),
            full((1, 1)),
        ],
        out_specs=pl.BlockSpec((_EB, 1), lambda i: (i, 0)),
        out_shape=jax.ShapeDtypeStruct((E, 1), jnp.float32),
        interpret=_INTERPRET,
    )(HI, HJ, CZI, CZJ, W1a, W1b, W1c, W1dp, be1[None, :], We2, be2[None, :],
      w3r, be3.reshape(1, 1))
    return out


def kernel(F_all, edge_und, edge_dir, Wp1, bp1, Wp2, bp2, Win, b_in,
           Ws0, bs0, Wn0, bn0, Ws1, bs1, Wn1, bn1, Ws2, bs2, Wn2, bn2,
           We1, be1, We2, be2, We3, be3):
    Wp1e = jnp.pad(Wp1, ((3, 0), (0, 0)))
    Wince = jnp.pad(Win[:3], ((0, D - 3), (0, 0)))
    Winp = Win[3:]
    W1a = We1[:H]
    W1b = We1[H:2 * H]
    W1c = We1[2 * H:3 * H]
    W1dp = jnp.pad(We1[3 * H:], ((0, 13), (0, 0)))
    w3r = We3[:, 0][None, :]

    czp, hs, hw = _node_stage(F_all, Wp1e, bp1, Wp2, bp2, Wince, Winp, b_in,
                              Ws0, bs0 + bn0, Wn0)

    src, dst = edge_dir[0], edge_dir[1]
    degv = jnp.zeros((N,), jnp.float32).at[dst].add(1.0)
    deg = jnp.broadcast_to(degv[:, None], (N, 16))

    seg = jnp.zeros((N, H), jnp.float32).at[dst].add(hw[src])
    hs, hw = _comb_stage(hs, seg, deg, Ws1, bs1 + bn1, Wn1)
    seg = jnp.zeros((N, H), jnp.float32).at[dst].add(hw[src])
    hs, hw = _comb_stage(hs, seg, deg, Ws2, bs2 + bn2, Wn2)
    seg = jnp.zeros((N, H), jnp.float32).at[dst].add(hw[src])
    h3 = _comb_final(hs, seg, deg)

    i, j = edge_und[0], edge_und[1]
    HI = h3[i]
    HJ = h3[j]
    CZI = czp[i]
    CZJ = czp[j]
    logits = _edge_stage(HI, HJ, CZI, CZJ, W1a, W1b, W1c, W1dp, be1,
                         We2, be2, w3r, be3)
    return logits.reshape(-1)


# speedup vs baseline: 2.6818x; 2.4761x over previous
import functools

import jax
import jax.numpy as jnp
from jax import lax
from jax.experimental import pallas as pl
from jax.experimental.pallas import tpu as pltpu
from jax.experimental.pallas import tpu_sc as plsc

N = 10000
E = 320000
D = 128
H = 128

_INTERPRET = False

_NC = 2
_NS = 16
_NW = _NC * _NS

_K = 128
_NCH = -(-E // (_NW * _K))
E_PAD = _NW * _K * _NCH
N_PAD = 10112
_RPT = N_PAD // _NS

_mesh = plsc.VectorSubcoreMesh(core_axis_name="c", subcore_axis_name="s",
                               num_cores=_NC, num_subcores=_NS)


def _dot(a, b):
    return jnp.dot(a, b, preferred_element_type=jnp.float32)


def _deg_body(dst_hbm, z128_hbm, ones_hbm, out0, out1, idx_d, ones_v, dacc):
    cid = lax.axis_index("c")
    sid = lax.axis_index("s")
    wid = sid * _NC + cid
    r0 = sid * _RPT
    pltpu.sync_copy(z128_hbm.at[pl.ds(r0, _RPT)], dacc.at[pl.ds(r0, _RPT)])
    pltpu.sync_copy(ones_hbm, ones_v)
    plsc.subcore_barrier()
    base0 = wid * (_K * _NCH)

    def chunk(c, carry):
        b = base0 + c * _K
        pltpu.sync_copy(dst_hbm.at[pl.ds(b, _K)], idx_d)
        pltpu.sync_copy(ones_v, dacc.at[idx_d], add=True)
        return carry

    lax.fori_loop(0, _NCH, chunk, 0)
    plsc.subcore_barrier()

    @pl.when(cid == 0)
    def _():
        pltpu.sync_copy(dacc.at[pl.ds(r0, _RPT)], out0.at[pl.ds(r0, _RPT)])

    @pl.when(cid == 1)
    def _():
        pltpu.sync_copy(dacc.at[pl.ds(r0, _RPT)], out1.at[pl.ds(r0, _RPT)])


_deg_kernel = pl.kernel(
    _deg_body,
    out_type=(
        jax.ShapeDtypeStruct((N_PAD, H), jnp.float32),
        jax.ShapeDtypeStruct((N_PAD, H), jnp.float32),
    ),
    mesh=_mesh,
    scratch_types=[
        pltpu.VMEM((_K,), jnp.int32),
        pltpu.VMEM((_K, H), jnp.float32),
        pltpu.VMEM_SHARED((N_PAD, H), jnp.float32),
    ],
)


def _seg_body(hw_hbm, src_hbm, dst_hbm, z128_hbm,
              out0, out1, idx_s, idx_d, rows_v, acc, sem):
    cid = lax.axis_index("c")
    sid = lax.axis_index("s")
    wid = sid * _NC + cid
    r0 = sid * _RPT
    pltpu.sync_copy(z128_hbm.at[pl.ds(r0, _RPT)], acc.at[pl.ds(r0, _RPT)])
    plsc.subcore_barrier()
    base0 = wid * (_K * _NCH)

    def chunk(c, carry):
        b = base0 + c * _K
        pltpu.sync_copy(src_hbm.at[pl.ds(b, _K)], idx_s)
        pltpu.sync_copy(dst_hbm.at[pl.ds(b, _K)], idx_d)
        pltpu.async_copy(hw_hbm.at[idx_s], rows_v, sem).wait()
        pltpu.sync_copy(rows_v, acc.at[idx_d], add=True)
        return carry

    lax.fori_loop(0, _NCH, chunk, 0)
    plsc.subcore_barrier()

    @pl.when(cid == 0)
    def _():
        pltpu.sync_copy(acc.at[pl.ds(r0, _RPT)], out0.at[pl.ds(r0, _RPT)])

    @pl.when(cid == 1)
    def _():
        pltpu.sync_copy(acc.at[pl.ds(r0, _RPT)], out1.at[pl.ds(r0, _RPT)])


_seg_kernel = pl.kernel(
    _seg_body,
    out_type=(
        jax.ShapeDtypeStruct((N_PAD, H), jnp.float32),
        jax.ShapeDtypeStruct((N_PAD, H), jnp.float32),
    ),
    mesh=_mesh,
    scratch_types=[
        pltpu.VMEM((_K,), jnp.int32),
        pltpu.VMEM((_K,), jnp.int32),
        pltpu.VMEM((_K, H), jnp.float32),
        pltpu.VMEM_SHARED((N_PAD, H), jnp.float32),
        pltpu.SemaphoreType.DMA,
    ],
)


def _egather_body(t_hbm, i_hbm, j_hbm, ti_out, tj_out,
                  idx_i, idx_j, bi_v, bj_v, sem):
    cid = lax.axis_index("c")
    sid = lax.axis_index("s")
    wid = sid * _NC + cid
    base0 = wid * (_K * _NCH)

    def chunk(c, carry):
        b = base0 + c * _K
        pltpu.sync_copy(i_hbm.at[pl.ds(b, _K)], idx_i)
        pltpu.sync_copy(j_hbm.at[pl.ds(b, _K)], idx_j)
        pltpu.async_copy(t_hbm.at[idx_i], bi_v, sem).wait()
        pltpu.async_copy(t_hbm.at[idx_j], bj_v, sem).wait()
        pltpu.sync_copy(bi_v, ti_out.at[pl.ds(b, _K)])
        pltpu.sync_copy(bj_v, tj_out.at[pl.ds(b, _K)])
        return carry

    lax.fori_loop(0, _NCH, chunk, 0)


_egather_kernel = pl.kernel(
    _egather_body,
    out_type=(
        jax.ShapeDtypeStruct((E_PAD, 2 * H), jnp.float32),
        jax.ShapeDtypeStruct((E_PAD, 2 * H), jnp.float32),
    ),
    mesh=_mesh,
    scratch_types=[
        pltpu.VMEM((_K,), jnp.int32),
        pltpu.VMEM((_K,), jnp.int32),
        pltpu.VMEM((_K, 2 * H), jnp.float32),
        pltpu.VMEM((_K, 2 * H), jnp.float32),
        pltpu.SemaphoreType.DMA,
    ],
)


def _node_body(f_ref, wp1_ref, bp1_ref, wp2_ref, bp2_ref, winc_ref, winp_ref,
               bin_ref, ws_ref, bsn_ref, wn_ref,
               czp_ref, hs_ref, hw_ref):
    x = f_ref[...]
    n = x.shape[0]
    col = lax.broadcasted_iota(jnp.int32, (1, D), 1)
    mask = col < 3
    s1 = jnp.sum(x, axis=0, keepdims=True)
    s2 = jnp.sum(x * x, axis=0, keepdims=True)
    m = s1 / n
    var = (s2 - n * m * m) / (n - 1)
    std = jnp.sqrt(jnp.maximum(var, 0.0))
    rs = jnp.where(mask, 1.0 / (std + 1e-6), 0.0)
    mm = jnp.where(mask, m, 0.0)
    czf = (x - mm) * rs
    p1 = jax.nn.relu(_dot(x, wp1_ref[...]) + bp1_ref[...])
    p2 = jax.nn.relu(_dot(p1, wp2_ref[...]) + bp2_ref[...])
    h0 = jax.nn.relu(_dot(czf, winc_ref[...]) + _dot(p2, winp_ref[...])
                     + bin_ref[...])
    czp_ref[...] = czf[:, :16]
    hs_ref[...] = _dot(h0, ws_ref[...]) + bsn_ref[...]
    hw_ref[...] = _dot(h0, wn_ref[...])


def _node_stage(F_all, Wp1e, bp1, Wp2, bp2, Wince, Winp, b_in, Ws, bsn, Wn):
    return pl.pallas_call(
        _node_body,
        out_shape=(
            jax.ShapeDtypeStruct((N, 16), jnp.float32),
            jax.ShapeDtypeStruct((N, H), jnp.float32),
            jax.ShapeDtypeStruct((N, H), jnp.float32),
        ),
        interpret=_INTERPRET,
    )(F_all, Wp1e, bp1[None, :], Wp2, bp2[None, :], Wince, Winp,
      b_in[None, :], Ws, bsn[None, :], Wn)


def _comb_body(hs_ref, sega_ref, segb_ref, dega_ref, degb_ref,
               ws_ref, bsn_ref, wn_ref, hs2_ref, hw2_ref):
    d = dega_ref[:N] + degb_ref[:N]
    inv = 1.0 / jnp.maximum(d, 1.0)
    seg = sega_ref[:N] + segb_ref[:N]
    h = jax.nn.relu(hs_ref[...] + seg * inv)
    hs2_ref[...] = _dot(h, ws_ref[...]) + bsn_ref[...]
    hw2_ref[...] = _dot(h, wn_ref[...])


def _comb_stage(hs, sega, segb, dega, degb, Ws, bsn, Wn):
    return pl.pallas_call(
        _comb_body,
        out_shape=(
            jax.ShapeDtypeStruct((N, H), jnp.float32),
            jax.ShapeDtypeStruct((N, H), jnp.float32),
        ),
        interpret=_INTERPRET,
    )(hs, sega, segb, dega, degb, Ws, bsn[None, :], Wn)


def _comb_final_body(hs_ref, sega_ref, segb_ref, dega_ref, degb_ref, h_ref):
    d = dega_ref[:N] + degb_ref[:N]
    inv = 1.0 / jnp.maximum(d, 1.0)
    seg = sega_ref[:N] + segb_ref[:N]
    h_ref[...] = jax.nn.relu(hs_ref[...] + seg * inv)


def _comb_final(hs, sega, segb, dega, degb):
    return pl.pallas_call(
        _comb_final_body,
        out_shape=jax.ShapeDtypeStruct((N, H), jnp.float32),
        interpret=_INTERPRET,
    )(hs, sega, segb, dega, degb)


_EB = 2000


def _edge_body(ti_ref, tj_ref, w1ac_ref, w1bc_ref, w1c_ref,
               be1_ref, w2_ref, be2_ref, w3_ref, be3_ref, out_ref):
    ti = ti_ref[...]
    tj = tj_ref[...]
    ab = jnp.abs(ti[:, :H] - tj[:, :H])
    x = _dot(ti, w1ac_ref[...]) + _dot(tj, w1bc_ref[...]) + _dot(ab, w1c_ref[...])
    x = jax.nn.relu(x + be1_ref[...])
    x = jax.nn.relu(_dot(x, w2_ref[...]) + be2_ref[...])
    out_ref[...] = jnp.sum(x * w3_ref[...], axis=1, keepdims=True) + be3_ref[...]


def _edge_stage(TI, TJ, W1ac, W1bc, W1c, be1, We2, be2, w3r, be3):
    grid = (E // _EB,)
    full = lambda shape: pl.BlockSpec(shape, lambda i: (0, 0))
    out = pl.pallas_call(
        _edge_body,
        grid=grid,
        in_specs=[
            pl.BlockSpec((_EB, 2 * H), lambda i: (i, 0)),
            pl.BlockSpec((_EB, 2 * H), lambda i: (i, 0)),
            full((2 * H, 256)), full((2 * H, 256)), full((H, 256)),
            full((1, 256)), full((256, 256)), full((1, 256)), full((1, 256)),
            full((1, 1)),
        ],
        out_specs=pl.BlockSpec((_EB, 1), lambda i: (i, 0)),
        out_shape=jax.ShapeDtypeStruct((E, 1), jnp.float32),
        interpret=_INTERPRET,
    )(TI, TJ, W1ac, W1bc, W1c, be1[None, :], We2, be2[None, :],
      w3r, be3.reshape(1, 1))
    return out


def kernel(F_all, edge_und, edge_dir, Wp1, bp1, Wp2, bp2, Win, b_in,
           Ws0, bs0, Wn0, bn0, Ws1, bs1, Wn1, bn1, Ws2, bs2, Wn2, bn2,
           We1, be1, We2, be2, We3, be3):
    f32 = jnp.float32
    Wp1e = jnp.pad(Wp1, ((3, 0), (0, 0)))
    Wince = jnp.pad(Win[:3], ((0, D - 3), (0, 0)))
    Winp = Win[3:]
    W1a = We1[:H]
    W1b = We1[H:2 * H]
    W1c = We1[2 * H:3 * H]
    W1d = We1[3 * H:]
    W1ac = jnp.concatenate([W1a, jnp.pad(W1d, ((0, H - 3), (0, 0)))], axis=0)
    W1bc = jnp.concatenate([W1b, jnp.pad(-W1d, ((0, H - 3), (0, 0)))], axis=0)
    w3r = We3[:, 0][None, :]

    src, dst = edge_dir[0], edge_dir[1]
    i, j = edge_und[0], edge_und[1]
    padz = jnp.zeros((E_PAD - E,), jnp.int32)
    srcp = jnp.concatenate([src, padz])
    dstp = jnp.concatenate([dst, jnp.full((E_PAD - E,), N, jnp.int32)])
    ip = jnp.concatenate([i, padz])
    jp = jnp.concatenate([j, padz])
    z128 = jnp.zeros((N_PAD, H), f32)
    ones128 = jnp.ones((_K, H), f32)

    czp, hs, hw = _node_stage(F_all, Wp1e, bp1, Wp2, bp2, Wince, Winp, b_in,
                              Ws0, bs0 + bn0, Wn0)

    dega, degb = _deg_kernel(dstp, z128, ones128)
    sega, segb = _seg_kernel(hw, srcp, dstp, z128)
    hs, hw = _comb_stage(hs, sega, segb, dega, degb, Ws1, bs1 + bn1, Wn1)
    sega, segb = _seg_kernel(hw, srcp, dstp, z128)
    hs, hw = _comb_stage(hs, sega, segb, dega, degb, Ws2, bs2 + bn2, Wn2)
    sega, segb = _seg_kernel(hw, srcp, dstp, z128)
    h3 = _comb_final(hs, sega, segb, dega, degb)

    T = jnp.concatenate([h3, czp, jnp.zeros((N, 2 * H - H - 16), f32)], axis=1)
    TI, TJ = _egather_kernel(T, ip, jp)
    logits = _edge_stage(TI, TJ, W1ac, W1bc, W1c, be1, We2, be2, w3r, be3)
    return logits.reshape(-1)
